# baseline (speedup 1.0000x reference)
import functools

import jax
import jax.numpy as jnp
from jax import lax
from jax.experimental import pallas as pl
from jax.experimental.pallas import tpu as pltpu
from jax.experimental.pallas import tpu_sc as plsc

B = 16384
NUM_FEAT = 32
D_MCC = 32
D_CTRY = 16
HID = 128
D_OTHER = 64

NC = 2
NS = 16
NW = NC * NS
BPW = B // NW
CHUNK = 128
NCH = BPW // CHUNK


def _sc_gather_body(idx_hbm, mcc_t, ctry_t, card_t, merch_t,
                    mcc_o, ctry_o, card_o, merch_o,
                    idx_v, mcc_v, ctry_v, card_v, merch_v, sem):
    wid = lax.axis_index("s") * NC + lax.axis_index("c")
    base = wid * BPW
    pltpu.sync_copy(idx_hbm.at[wid], idx_v)
    copies = []
    for j in range(NCH):
        dst = pl.ds(j * CHUNK, CHUNK)
        copies.append(pltpu.async_copy(mcc_t.at[idx_v.at[0, j]], mcc_v.at[dst], sem))
        copies.append(pltpu.async_copy(ctry_t.at[idx_v.at[1, j]], ctry_v.at[dst], sem))
        copies.append(pltpu.async_copy(card_t.at[idx_v.at[2, j]], card_v.at[dst], sem))
        copies.append(pltpu.async_copy(merch_t.at[idx_v.at[3, j]], merch_v.at[dst], sem))
    for c in copies:
        c.wait()
    out = pl.ds(base, BPW)
    pltpu.sync_copy(mcc_v, mcc_o.at[out])
    pltpu.sync_copy(ctry_v, ctry_o.at[out])
    pltpu.sync_copy(card_v, card_o.at[out])
    pltpu.sync_copy(merch_v, merch_o.at[out])


@jax.jit
def _sc_gather(idx_packed, emb_mcc, emb_country, emb_card, emb_merchant):
    mesh = plsc.VectorSubcoreMesh(core_axis_name="c", subcore_axis_name="s",
                                  num_cores=NC, num_subcores=NS)
    f = pl.kernel(
        _sc_gather_body,
        out_type=(
            jax.ShapeDtypeStruct((B, D_MCC), jnp.float32),
            jax.ShapeDtypeStruct((B, D_CTRY), jnp.float32),
            jax.ShapeDtypeStruct((B, D_OTHER), jnp.float32),
            jax.ShapeDtypeStruct((B, D_OTHER), jnp.float32),
        ),
        mesh=mesh,
        scratch_types=[
            pltpu.VMEM((4, NCH, CHUNK), jnp.int32),
            pltpu.VMEM((BPW, D_MCC), jnp.float32),
            pltpu.VMEM((BPW, D_CTRY), jnp.float32),
            pltpu.VMEM((BPW, D_OTHER), jnp.float32),
            pltpu.VMEM((BPW, D_OTHER), jnp.float32),
            pltpu.SemaphoreType.DMA,
        ],
        compiler_params=pltpu.CompilerParams(use_tc_tiling_on_sc=False),
    )
    return f(idx_packed, emb_mcc, emb_country, emb_card, emb_merchant)


BT = 2048


def _tc_body(xn, em, ec, cr, mr, wt, bt, wc, bc, wm, bm, ot, oc, om):
    t = jnp.dot(xn[...], wt[0:NUM_FEAT, :], preferred_element_type=jnp.float32)
    t = t + jnp.dot(em[...], wt[NUM_FEAT:NUM_FEAT + D_MCC, :],
                    preferred_element_type=jnp.float32)
    t = t + jnp.dot(ec[...], wt[NUM_FEAT + D_MCC:, :],
                    preferred_element_type=jnp.float32)
    ot[...] = t + bt[...]
    oc[...] = jnp.dot(cr[...], wc[...], preferred_element_type=jnp.float32) + bc[...]
    om[...] = jnp.dot(mr[...], wm[...], preferred_element_type=jnp.float32) + bm[...]


@jax.jit
def _tc_project(x_num, e_mcc, e_ctry, card_rows, merch_rows,
                W_trans, b_trans, W_card, b_card, W_merchant, b_merchant):
    row = lambda d: pl.BlockSpec((BT, d), lambda i: (i, 0))
    full = lambda a: pl.BlockSpec(a.shape, lambda i: (0,) * a.ndim)
    return pl.pallas_call(
        _tc_body,
        grid=(B // BT,),
        in_specs=[row(NUM_FEAT), row(D_MCC), row(D_CTRY), row(D_OTHER),
                  row(D_OTHER), full(W_trans), full(b_trans), full(W_card),
                  full(b_card), full(W_merchant), full(b_merchant)],
        out_specs=[row(HID), row(HID), row(HID)],
        out_shape=[jax.ShapeDtypeStruct((B, HID), jnp.float32)] * 3,
    )(x_num, e_mcc, e_ctry, card_rows, merch_rows,
      W_trans, b_trans, W_card, b_card, W_merchant, b_merchant)


def kernel(x_num, x_cat, n_id_card, n_id_merchant,
           emb_mcc, emb_country, W_trans, b_trans,
           emb_card, W_card, b_card,
           emb_merchant, W_merchant, b_merchant):
    idx_mcc = jnp.clip(x_cat[:, 0] + 1, 0, emb_mcc.shape[0] - 1).astype(jnp.int32)
    idx_ctry = jnp.clip(x_cat[:, 1] + 1, 0, emb_country.shape[0] - 1).astype(jnp.int32)
    idx_packed = jnp.stack(
        [idx_mcc.reshape(NW, BPW), idx_ctry.reshape(NW, BPW),
         n_id_card.reshape(NW, BPW), n_id_merchant.reshape(NW, BPW)],
        axis=1).reshape(NW, 4, NCH, CHUNK)
    e_mcc, e_ctry, card_rows, merch_rows = _sc_gather(
        idx_packed, emb_mcc, emb_country, emb_card, emb_merchant)
    out_trans, out_card, out_merch = _tc_project(
        x_num, e_mcc, e_ctry, card_rows, merch_rows,
        W_trans, b_trans.reshape(1, HID), W_card, b_card.reshape(1, HID),
        W_merchant, b_merchant.reshape(1, HID))
    return (out_trans, out_card, out_merch)


# speedup vs baseline: 1.5248x; 1.5248x over previous
import jax
import jax.numpy as jnp
from jax import lax
from jax.experimental import pallas as pl
from jax.experimental.pallas import tpu as pltpu
from jax.experimental.pallas import tpu_sc as plsc

B = 16384
NUM_FEAT = 32
D_MCC = 32
D_CTRY = 16
HID = 128
D_OTHER = 64

NC = 2
NS = 16
NW = NC * NS
BPW = B // NW

GRP = 32
NGRP = BPW // GRP

CHS = 128
NCHS = BPW // CHS


def _sc_big_body(idx_hbm, card_t, merch_t, card_o, merch_o,
                 idx_v, rows_v, sem):
    wid = lax.axis_index("s") * NC + lax.axis_index("c")
    base = wid * BPW
    pltpu.sync_copy(idx_hbm.at[wid], idx_v)
    for k, (tbl, out) in enumerate(((card_t, card_o), (merch_t, merch_o))):

        @pl.loop(0, NGRP)
        def _grp(g, _tbl=tbl, _k=k):
            r0 = g * GRP
            cps = []
            for h in range(GRP // 16):
                v16 = idx_v[_k, pl.ds(r0 + 16 * h, 16)]
                for j in range(16):
                    cps.append(pltpu.async_copy(
                        _tbl.at[v16[j]], rows_v.at[r0 + 16 * h + j], sem))
            for cp in cps:
                cp.wait()

        pltpu.sync_copy(rows_v, out.at[pl.ds(base, BPW)])


@jax.jit
def _sc_big(idx_packed, card_t, merch_t):
    mesh = plsc.VectorSubcoreMesh(core_axis_name="c", subcore_axis_name="s",
                                  num_cores=NC, num_subcores=NS)
    f = pl.kernel(
        _sc_big_body,
        out_type=(
            jax.ShapeDtypeStruct((B, D_OTHER), jnp.float32),
            jax.ShapeDtypeStruct((B, D_OTHER), jnp.float32),
        ),
        mesh=mesh,
        scratch_types=[
            pltpu.VMEM((2, BPW), jnp.int32),
            pltpu.VMEM((BPW, D_OTHER), jnp.float32),
            pltpu.SemaphoreType.DMA,
        ],
    )
    return f(idx_packed, card_t, merch_t)


def _sc_small_body(idx_hbm, mcc_t, ctry_t, mcc_o, ctry_o,
                   idx_v, mcc_v, ctry_v, sem):
    wid = lax.axis_index("s") * NC + lax.axis_index("c")
    base = wid * BPW
    pltpu.sync_copy(idx_hbm.at[wid], idx_v)
    copies = []
    for j in range(NCHS):
        dst = pl.ds(j * CHS, CHS)
        copies.append(pltpu.async_copy(mcc_t.at[idx_v.at[0, j]], mcc_v.at[dst], sem))
        copies.append(pltpu.async_copy(ctry_t.at[idx_v.at[1, j]], ctry_v.at[dst], sem))
    for c in copies:
        c.wait()
    out = pl.ds(base, BPW)
    pltpu.sync_copy(mcc_v, mcc_o.at[out])
    pltpu.sync_copy(ctry_v, ctry_o.at[out])


@jax.jit
def _sc_small(idx_packed, emb_mcc, emb_country):
    mesh = plsc.VectorSubcoreMesh(core_axis_name="c", subcore_axis_name="s",
                                  num_cores=NC, num_subcores=NS)
    f = pl.kernel(
        _sc_small_body,
        out_type=(
            jax.ShapeDtypeStruct((B, D_MCC), jnp.float32),
            jax.ShapeDtypeStruct((B, D_CTRY), jnp.float32),
        ),
        mesh=mesh,
        scratch_types=[
            pltpu.VMEM((2, NCHS, CHS), jnp.int32),
            pltpu.VMEM((BPW, D_MCC), jnp.float32),
            pltpu.VMEM((BPW, D_CTRY), jnp.float32),
            pltpu.SemaphoreType.DMA,
        ],
        compiler_params=pltpu.CompilerParams(use_tc_tiling_on_sc=False),
    )
    return f(idx_packed, emb_mcc, emb_country)


BT = 2048


def _tc_body(xn, em, ec, cr, mr, wt, bt, wc, bc, wm, bm, ot, oc, om):
    t = jnp.dot(xn[...], wt[0:NUM_FEAT, :], preferred_element_type=jnp.float32)
    t = t + jnp.dot(em[...], wt[NUM_FEAT:NUM_FEAT + D_MCC, :],
                    preferred_element_type=jnp.float32)
    t = t + jnp.dot(ec[...], wt[NUM_FEAT + D_MCC:, :],
                    preferred_element_type=jnp.float32)
    ot[...] = t + bt[...]
    oc[...] = jnp.dot(cr[...], wc[...], preferred_element_type=jnp.float32) + bc[...]
    om[...] = jnp.dot(mr[...], wm[...], preferred_element_type=jnp.float32) + bm[...]


@jax.jit
def _tc_project(x_num, e_mcc, e_ctry, card_rows, merch_rows,
                W_trans, b_trans, W_card, b_card, W_merchant, b_merchant):
    row = lambda d: pl.BlockSpec((BT, d), lambda i: (i, 0))
    full = lambda a: pl.BlockSpec(a.shape, lambda i: (0,) * a.ndim)
    return pl.pallas_call(
        _tc_body,
        grid=(B // BT,),
        in_specs=[row(NUM_FEAT), row(D_MCC), row(D_CTRY), row(D_OTHER),
                  row(D_OTHER), full(W_trans), full(b_trans), full(W_card),
                  full(b_card), full(W_merchant), full(b_merchant)],
        out_specs=[row(HID), row(HID), row(HID)],
        out_shape=[jax.ShapeDtypeStruct((B, HID), jnp.float32)] * 3,
    )(x_num, e_mcc, e_ctry, card_rows, merch_rows,
      W_trans, b_trans, W_card, b_card, W_merchant, b_merchant)


def kernel(x_num, x_cat, n_id_card, n_id_merchant,
           emb_mcc, emb_country, W_trans, b_trans,
           emb_card, W_card, b_card,
           emb_merchant, W_merchant, b_merchant):
    idx_mcc = jnp.clip(x_cat[:, 0] + 1, 0, emb_mcc.shape[0] - 1).astype(jnp.int32)
    idx_ctry = jnp.clip(x_cat[:, 1] + 1, 0, emb_country.shape[0] - 1).astype(jnp.int32)
    idx_small = jnp.stack(
        [idx_mcc.reshape(NW, BPW), idx_ctry.reshape(NW, BPW)],
        axis=1).reshape(NW, 2, NCHS, CHS)
    idx_big = jnp.stack(
        [n_id_card.reshape(NW, BPW), n_id_merchant.reshape(NW, BPW)], axis=1)
    card_rows, merch_rows = _sc_big(idx_big, emb_card, emb_merchant)
    e_mcc, e_ctry = _sc_small(idx_small, emb_mcc, emb_country)
    out_trans, out_card, out_merch = _tc_project(
        x_num, e_mcc, e_ctry, card_rows, merch_rows,
        W_trans, b_trans.reshape(1, HID), W_card, b_card.reshape(1, HID),
        W_merchant, b_merchant.reshape(1, HID))
    return (out_trans, out_card, out_merch)
